# baseline (speedup 1.0000x reference)
import functools

import jax
import jax.numpy as jnp
from jax import lax
from jax.experimental import pallas as pl
from jax.experimental.pallas import tpu as pltpu
from jax.experimental.pallas import tpu_sc as plsc

_CB = 6

_B, _C, _H, _W = 2, 96, 512, 512
_HO, _WO = _H // 2 + 2, _W // 2 + 2


def _sc_fill_body(x_hbm, out_hbm, xv, pairv):
    wid = lax.axis_index("s") * 2 + lax.axis_index("c")

    def plane_step(t, carry):
        p = wid * 6 + t
        b = p // _C
        c = p % _C
        pltpu.sync_copy(x_hbm.at[b, c, :, pl.ds(384, 128)], xv)
        for s1 in range(2):
            for s2 in range(2):
                ch = c * 4 + 2 * s1 + s2
                for k in range(17):
                    ho = lax.iota(jnp.int32, 16) + k * 16
                    hsrc = 2 * jnp.clip(ho - 1, 0, 255) + s1
                    col = jnp.full((16,), 126 + s2, jnp.int32)
                    val = plsc.load_gather(xv, [hsrc, col])
                    plsc.store_scatter(pairv, [ho, jnp.zeros((16,), jnp.int32)], val)
                    plsc.store_scatter(pairv, [ho, jnp.ones((16,), jnp.int32)], val)
                pltpu.sync_copy(
                    pairv.at[pl.ds(0, _HO)],
                    out_hbm.at[b, ch, :, pl.ds(_WO - 2, 2)],
                )
        return carry

    lax.fori_loop(0, 6, plane_step, 0)


def _sc_fill(x):
    mesh = plsc.VectorSubcoreMesh(core_axis_name="c", subcore_axis_name="s")
    fn = pl.kernel(
        _sc_fill_body,
        out_type=jax.ShapeDtypeStruct((_B, 4 * _C, _HO, _WO), jnp.float32),
        mesh=mesh,
        scratch_types=[
            pltpu.VMEM((_H, 128), jnp.float32),
            pltpu.VMEM((272, 2), jnp.float32),
        ],
        compiler_params=pltpu.CompilerParams(
            use_tc_tiling_on_sc=True, needs_layout_passes=False
        ),
    )
    return fn(x)


def _ta(arr, idx):
    return jnp.take_along_axis(arr, idx, axis=1)


def _tc_bulk_kernel(x_ref, s_ref, o_ref):
    del s_ref
    ii = jax.lax.broadcasted_iota(jnp.int32, (256, 256), 0)
    jj = jax.lax.broadcasted_iota(jnp.int32, (256, 256), 1)
    D = (ii == 2 * (jj % 128) + jj // 128).astype(jnp.bfloat16)
    for ci in range(_CB):
        _one_plane(x_ref[0, ci], o_ref.at[0, 4 * ci : 4 * ci + 4], D)


def _one_plane(x, o_ref, D):
    xh = x.astype(jnp.bfloat16)
    xl = (x - xh.astype(jnp.float32)).astype(jnp.bfloat16)
    y = jnp.concatenate(
        [
            jnp.dot(
                xh[:, h * 256 : (h + 1) * 256],
                D,
                preferred_element_type=jnp.float32,
            )
            + jnp.dot(
                xl[:, h * 256 : (h + 1) * 256],
                D,
                preferred_element_type=jnp.float32,
            )
            for h in range(2)
        ],
        axis=1,
    )
    y4 = y.reshape(32, 2, 8, 512)
    ye = y4[:, 0]
    yo = y4[:, 1]
    yp = jnp.roll(yo, 1, axis=0)
    si = jax.lax.broadcasted_iota(jnp.int32, (32, 8, 512), 1)
    row = jax.lax.broadcasted_iota(jnp.int32, (256, 512), 0)
    for s1 in range(2):
        q = (2 * si - 2 + s1) % 8
        g = jnp.where(
            si == 0,
            _ta(yp, q),
            jnp.where(si <= 4, _ta(ye, q), _ta(yo, q)),
        ).reshape(256, 512)
        g = jnp.where(row == 0, jnp.broadcast_to(y[s1 : s1 + 1, :], (256, 512)), g)
        gt = jnp.broadcast_to(y[510 + s1 : 511 + s1, :], (2, 512))
        z = jnp.concatenate([g, gt], axis=0)
        for s2 in range(2):
            core = jnp.concatenate(
                [
                    z[:, 128 * s2 : 128 * s2 + 128],
                    z[:, 256 + 128 * s2 : 256 + 128 * s2 + 128],
                ],
                axis=1,
            )
            full = jnp.concatenate([core[:, 0:1], core[:, 0:255]], axis=1)
            o_ref[2 * s1 + s2] = full


def _tc_bulk(x, s):
    return pl.pallas_call(
        _tc_bulk_kernel,
        grid=(_B, _C // _CB),
        in_specs=[
            pl.BlockSpec((1, _CB, _H, _W), lambda b, c: (b, c, 0, 0)),
            pl.BlockSpec(memory_space=pl.MemorySpace.ANY),
        ],
        out_specs=pl.BlockSpec(
            (1, 4 * _CB, _HO, 256), lambda b, c: (b, c, 0, 0)
        ),
        out_shape=jax.ShapeDtypeStruct((_B, 4 * _C, _HO, _WO), jnp.float32),
        input_output_aliases={1: 0},
        compiler_params=pltpu.CompilerParams(
            dimension_semantics=("parallel", "parallel"),
        ),
    )(x, s)


def kernel(x):
    s = _sc_fill(x)
    return _tc_bulk(x, s)


# speedup vs baseline: 1.1220x; 1.1220x over previous
import jax
import jax.numpy as jnp
from jax.experimental import pallas as pl
from jax.experimental.pallas import tpu as pltpu


def _ta(arr, idx):
    return jnp.take_along_axis(arr, idx, axis=1)


_CB = 6


def _unshuffle_pad_kernel(x_ref, o_ref):
    ii = jax.lax.broadcasted_iota(jnp.int32, (256, 256), 0)
    jj = jax.lax.broadcasted_iota(jnp.int32, (256, 256), 1)
    D = (ii == 2 * (jj % 128) + jj // 128).astype(jnp.bfloat16)
    for ci in range(_CB):
        _one_plane(x_ref[0, ci], o_ref.at[0, 4 * ci : 4 * ci + 4], D)


def _one_plane(x, o_ref, D):
    xh = x.astype(jnp.bfloat16)
    xl = (x - xh.astype(jnp.float32)).astype(jnp.bfloat16)
    y = jnp.concatenate(
        [
            jnp.dot(
                xh[:, h * 256 : (h + 1) * 256],
                D,
                preferred_element_type=jnp.float32,
            )
            + jnp.dot(
                xl[:, h * 256 : (h + 1) * 256],
                D,
                preferred_element_type=jnp.float32,
            )
            for h in range(2)
        ],
        axis=1,
    )
    y4 = y.reshape(32, 2, 8, 512)
    ye = y4[:, 0]
    yo = y4[:, 1]
    yp = jnp.roll(yo, 1, axis=0)
    si = jax.lax.broadcasted_iota(jnp.int32, (32, 8, 512), 1)
    row = jax.lax.broadcasted_iota(jnp.int32, (256, 512), 0)
    for s1 in range(2):
        q = (2 * si - 2 + s1) % 8
        g = jnp.where(
            si == 0,
            _ta(yp, q),
            jnp.where(si <= 4, _ta(ye, q), _ta(yo, q)),
        ).reshape(256, 512)
        g = jnp.where(row == 0, jnp.broadcast_to(y[s1 : s1 + 1, :], (256, 512)), g)
        gt = jnp.broadcast_to(y[510 + s1 : 511 + s1, :], (2, 512))
        z = jnp.concatenate([g, gt], axis=0)
        for s2 in range(2):
            core = jnp.concatenate(
                [
                    z[:, 128 * s2 : 128 * s2 + 128],
                    z[:, 256 + 128 * s2 : 256 + 128 * s2 + 128],
                ],
                axis=1,
            )
            full = jnp.concatenate(
                [core[:, 0:1], core, core[:, 255:256]], axis=1
            )
            o_ref[2 * s1 + s2] = full


def kernel(x):
    B, C, H, W = x.shape
    Ho, Wo = H // 2 + 2, W // 2 + 2
    return pl.pallas_call(
        _unshuffle_pad_kernel,
        grid=(B, C // _CB),
        in_specs=[pl.BlockSpec((1, _CB, H, W), lambda b, c: (b, c, 0, 0))],
        out_specs=pl.BlockSpec((1, 4 * _CB, Ho, Wo), lambda b, c: (b, c, 0, 0)),
        out_shape=jax.ShapeDtypeStruct((B, 4 * C, Ho, Wo), x.dtype),
        compiler_params=pltpu.CompilerParams(
            dimension_semantics=("parallel", "parallel"),
        ),
    )(x)
